# baseline (speedup 1.0000x reference)
import functools

import jax
import jax.numpy as jnp
from jax import lax
from jax.experimental import pallas as pl
from jax.experimental.pallas import tpu as pltpu
from jax.experimental.pallas import tpu_sc as plsc

NC = 2
NS = 16
NW = NC * NS
LANES = 16


def _mesh():
    return plsc.VectorSubcoreMesh(core_axis_name="c", subcore_axis_name="s",
                                  num_cores=NC, num_subcores=NS)


@functools.cache
def _build_deg_kernel(NP, C, K):
    rps = NP // NS
    ZR = rps // 5

    @functools.partial(
        pl.kernel,
        mesh=_mesh(),
        compiler_params=pltpu.CompilerParams(use_tc_tiling_on_sc=False),
        out_type=jax.ShapeDtypeStruct((NC, NP, LANES), jnp.float32),
        scratch_types=[
            pltpu.VMEM((C, K), jnp.int32),
            pltpu.VMEM((K, LANES), jnp.float32),
            pltpu.VMEM((ZR, LANES), jnp.float32),
            pltpu.VMEM_SHARED((NP, LANES), jnp.float32),
        ],
    )
    def deg_kernel(dst_hbm, out_hbm, dst_v, ones_v, z_v, acc_sh):
        cid = lax.axis_index("c")
        sid = lax.axis_index("s")
        wid = cid * NS + sid

        def _fill(i, carry):
            ones_v[i] = jnp.ones((LANES,), jnp.float32)
            return carry
        lax.fori_loop(0, K, _fill, 0)

        def _zrow(i, carry):
            z_v[i] = jnp.zeros((LANES,), jnp.float32)
            return carry
        lax.fori_loop(0, ZR, _zrow, 0)

        def _zcopy(j, carry):
            pltpu.sync_copy(z_v, acc_sh.at[pl.ds(sid * rps + j * ZR, ZR)])
            return carry
        lax.fori_loop(0, rps // ZR, _zcopy, 0)
        plsc.subcore_barrier()

        pltpu.sync_copy(dst_hbm.at[wid], dst_v)

        def _chunk(j, carry):
            pltpu.sync_copy(ones_v, acc_sh.at[dst_v.at[j]], add=True)
            return carry
        lax.fori_loop(0, C, _chunk, 0)
        plsc.subcore_barrier()

        pltpu.sync_copy(acc_sh.at[pl.ds(sid * rps, rps)],
                        out_hbm.at[cid, pl.ds(sid * rps, rps)])

    return deg_kernel


@functools.cache
def _build_edge_kernel(NP, H, C, K, NB=5):
    rps = NP // NS
    ZR = rps // 5
    assert C % NB == 0

    @functools.partial(
        pl.kernel,
        mesh=_mesh(),
        compiler_params=pltpu.CompilerParams(use_tc_tiling_on_sc=False),
        out_type=jax.ShapeDtypeStruct((NC, NP, H), jnp.float32),
        scratch_types=[
            pltpu.VMEM((C, K), jnp.int32),
            pltpu.VMEM((C, K), jnp.int32),
            pltpu.VMEM((NB, K, H), jnp.float32),
            pltpu.VMEM((ZR, H), jnp.float32),
            pltpu.VMEM_SHARED((NP, H), jnp.float32),
        ] + [pltpu.SemaphoreType.DMA] * NB,
    )
    def edge_kernel(g_hbm, src_hbm, dst_hbm, out_hbm,
                    src_v, dst_v, rows_v, z_v, acc_sh, *gsem):
        cid = lax.axis_index("c")
        sid = lax.axis_index("s")
        wid = cid * NS + sid

        def _zrow(i, carry):
            for j in range(H // LANES):
                z_v[i, pl.ds(j * LANES, LANES)] = jnp.zeros((LANES,), jnp.float32)
            return carry
        lax.fori_loop(0, ZR, _zrow, 0)

        def _zcopy(j, carry):
            pltpu.sync_copy(z_v, acc_sh.at[pl.ds(sid * rps + j * ZR, ZR)])
            return carry
        lax.fori_loop(0, rps // ZR, _zcopy, 0)
        plsc.subcore_barrier()

        pltpu.sync_copy(src_hbm.at[wid], src_v)
        pltpu.sync_copy(dst_hbm.at[wid], dst_v)

        for b in range(NB):
            pltpu.async_copy(g_hbm.at[src_v.at[b]], rows_v.at[b], gsem[b])

        def _iter(t, carry):
            for b in range(NB):
                j = t * NB + b
                pltpu.make_async_copy(g_hbm.at[src_v.at[j]],
                                      rows_v.at[b], gsem[b]).wait()
                pltpu.sync_copy(rows_v.at[b], acc_sh.at[dst_v.at[j]], add=True)
                nxt = j + NB

                @pl.when(nxt < C)
                def _():
                    pltpu.async_copy(g_hbm.at[src_v.at[nxt]],
                                     rows_v.at[b], gsem[b])
            return carry
        lax.fori_loop(0, C // NB, _iter, 0)
        plsc.subcore_barrier()

        pltpu.sync_copy(acc_sh.at[pl.ds(sid * rps, rps)],
                        out_hbm.at[cid, pl.ds(sid * rps, rps)])

    return edge_kernel


@functools.cache
def _build_pool_kernel(N, H, G, PW):
    Gp = G + 1

    @functools.partial(
        pl.kernel,
        mesh=_mesh(),
        compiler_params=pltpu.CompilerParams(use_tc_tiling_on_sc=False),
        out_type=(
            jax.ShapeDtypeStruct((NW, G, 2 * H), jnp.float32),
            jax.ShapeDtypeStruct((NW, G, LANES), jnp.float32),
        ),
        scratch_types=[
            pltpu.VMEM((PW, H), jnp.float32),
            pltpu.VMEM((PW,), jnp.int32),
            pltpu.VMEM((Gp, 2 * H), jnp.float32),
            pltpu.VMEM((Gp, LANES), jnp.float32),
        ],
    )
    def pool_kernel(h_hbm, bids_hbm, oms_hbm, ocnt_hbm,
                    rows_v, bids_v, ms_v, ct_v):
        cid = lax.axis_index("c")
        sid = lax.axis_index("s")
        wid = cid * NS + sid

        def _init(i, carry):
            for j in range(H // LANES):
                ms_v[i, pl.ds(j * LANES, LANES)] = jnp.full((LANES,), -jnp.inf,
                                                            jnp.float32)
                ms_v[i, pl.ds(H + j * LANES, LANES)] = jnp.zeros((LANES,),
                                                                 jnp.float32)
            ct_v[i] = jnp.zeros((LANES,), jnp.float32)
            return carry
        lax.fori_loop(0, Gp, _init, 0)

        pltpu.sync_copy(h_hbm.at[pl.ds(wid * PW, PW)], rows_v)
        pltpu.sync_copy(bids_hbm.at[wid], bids_v)

        def _grp(t, carry):
            bid16 = bids_v[pl.ds(t * LANES, LANES)]
            for j in range(LANES):
                bid = bid16[j]
                k = t * LANES + j
                for q in range(H // LANES):
                    r = rows_v[k, pl.ds(q * LANES, LANES)]
                    m = ms_v[bid, pl.ds(q * LANES, LANES)]
                    ms_v[bid, pl.ds(q * LANES, LANES)] = jnp.maximum(m, r)
                    s = ms_v[bid, pl.ds(H + q * LANES, LANES)]
                    ms_v[bid, pl.ds(H + q * LANES, LANES)] = s + r
                c = ct_v[bid]
                ct_v[bid] = c + 1.0
            return carry
        lax.fori_loop(0, PW // LANES, _grp, 0)

        pltpu.sync_copy(ms_v.at[pl.ds(0, G)], oms_hbm.at[wid])
        pltpu.sync_copy(ct_v.at[pl.ds(0, G)], ocnt_hbm.at[wid])

    return pool_kernel


def _tc_mm(X2, W02, BN2=640):
    NP2, F2 = X2.shape
    H2 = W02.shape[1]

    def body(x_ref, w_ref, t_ref):
        t_ref[...] = jnp.dot(x_ref[...], w_ref[...],
                             preferred_element_type=jnp.float32)

    return pl.pallas_call(
        body,
        grid=(NP2 // BN2,),
        in_specs=[
            pl.BlockSpec((BN2, F2), lambda i: (i, 0)),
            pl.BlockSpec((F2, H2), lambda i: (0, 0)),
        ],
        out_specs=pl.BlockSpec((BN2, H2), lambda i: (i, 0)),
        out_shape=jax.ShapeDtypeStruct((NP2, H2), jnp.float32),
    )(X2, W02)


def _tc_scale(t2, degp2, BN2=640):
    NP2, H2 = t2.shape

    def body(t_ref, dp_ref, g_ref, dv_ref):
        de = lax.rsqrt(jnp.maximum(
            1.0 + dp_ref[0, :, 0:1] + dp_ref[1, :, 0:1], 1.0))
        do = lax.rsqrt(jnp.maximum(
            1.0 + dp_ref[0, :, 16:17] + dp_ref[1, :, 16:17], 1.0))
        dv = jnp.concatenate(
            [jnp.broadcast_to(de, (BN2, H2 // 2)),
             jnp.broadcast_to(do, (BN2, H2 // 2))], axis=1)
        dv_ref[...] = dv
        g_ref[...] = t_ref[...] * dv

    return pl.pallas_call(
        body,
        grid=(NP2 // BN2,),
        in_specs=[
            pl.BlockSpec((BN2, H2), lambda i: (i, 0)),
            pl.BlockSpec((NC, BN2, 2 * LANES), lambda i: (0, i, 0)),
        ],
        out_specs=[
            pl.BlockSpec((BN2, H2), lambda i: (i, 0)),
            pl.BlockSpec((BN2, H2), lambda i: (i, 0)),
        ],
        out_shape=[
            jax.ShapeDtypeStruct((NP2, H2), jnp.float32),
            jax.ShapeDtypeStruct((NP2, H2), jnp.float32),
        ],
    )(t2, degp2)


def _tc_layer(A2, g2, dv2, b2, W2n, BN2=640):
    NP2, H2 = g2.shape

    def body(a_ref, g_ref, dv_ref, b_ref, w_ref, o_ref):
        dv = dv_ref[...]
        h = jnp.tanh(dv * (a_ref[0] + a_ref[1] + g_ref[...]) + b_ref[...])
        o_ref[...] = jnp.dot(h, w_ref[...],
                             preferred_element_type=jnp.float32) * dv

    return pl.pallas_call(
        body,
        grid=(NP2 // BN2,),
        in_specs=[
            pl.BlockSpec((NC, BN2, H2), lambda i: (0, i, 0)),
            pl.BlockSpec((BN2, H2), lambda i: (i, 0)),
            pl.BlockSpec((BN2, H2), lambda i: (i, 0)),
            pl.BlockSpec((1, H2), lambda i: (0, 0)),
            pl.BlockSpec((H2, H2), lambda i: (0, 0)),
        ],
        out_specs=pl.BlockSpec((BN2, H2), lambda i: (i, 0)),
        out_shape=jax.ShapeDtypeStruct((NP2, H2), jnp.float32),
    )(A2, g2, dv2, b2, W2n)


def _tc_last(A2, g2, dv2, b2, BN2=640):
    NP2, H2 = g2.shape

    def body(a_ref, g_ref, dv_ref, b_ref, o_ref):
        o_ref[...] = jnp.tanh(
            dv_ref[...] * (a_ref[0] + a_ref[1] + g_ref[...]) + b_ref[...])

    return pl.pallas_call(
        body,
        grid=(NP2 // BN2,),
        in_specs=[
            pl.BlockSpec((NC, BN2, H2), lambda i: (0, i, 0)),
            pl.BlockSpec((BN2, H2), lambda i: (i, 0)),
            pl.BlockSpec((BN2, H2), lambda i: (i, 0)),
            pl.BlockSpec((1, H2), lambda i: (0, 0)),
        ],
        out_specs=pl.BlockSpec((BN2, H2), lambda i: (i, 0)),
        out_shape=jax.ShapeDtypeStruct((NP2, H2), jnp.float32),
    )(A2, g2, dv2, b2)


def _tc_readout(msp, cntp, Wout, bout):
    G = msp.shape[1]
    H = msp.shape[2] // 2

    def body(ms_ref, ct_ref, w_ref, b_ref, out_ref, hid_ref):
        mx = jnp.max(ms_ref[:, :, :H], axis=0)
        mx = jnp.where(mx == -jnp.inf, 0.0, mx)
        sm = jnp.sum(ms_ref[:, :, H:], axis=0)
        ct = jnp.sum(ct_ref[:, :, 0], axis=0)
        mean = sm / jnp.maximum(ct, 1.0)[:, None]
        hidden = jnp.concatenate([mx, mean], axis=1)
        hid_ref[...] = hidden
        out_ref[...] = jnp.dot(hidden, w_ref[...],
                               preferred_element_type=jnp.float32) + b_ref[...]

    return pl.pallas_call(
        body,
        out_shape=(
            jax.ShapeDtypeStruct((G, 1), jnp.float32),
            jax.ShapeDtypeStruct((G, 2 * H), jnp.float32),
        ),
    )(msp, cntp, Wout, bout.reshape(1, 1))


def _pair_blockdiag(W):
    Fi, Fo = W.shape
    Z = jnp.zeros((Fi, Fo), W.dtype)
    return jnp.concatenate(
        [jnp.concatenate([W, Z], axis=1), jnp.concatenate([Z, W], axis=1)],
        axis=0)


def kernel(x, edge_index, batch_index, W0, b0, W1, b1, W2, b2, W3, b3,
           Wout, bout):
    N, F = x.shape
    H = W0.shape[1]
    E = edge_index.shape[1]
    G = 256
    K = 125
    EW = E // NW
    C = EW // K
    assert EW * NW == E and C * K == EW

    src3 = edge_index[0].reshape(NW, C, K)
    dst3 = edge_index[1].reshape(NW, C, K)

    NP = ((N + 1279) // 1280) * 1280
    deg_k = _build_deg_kernel(NP, C, K)
    edge_k = _build_edge_kernel(NP, H, C, K)

    degp = deg_k(dst3)
    degp2 = degp.reshape(NC, NP // 2, 2 * LANES)
    X2 = jnp.pad(x, ((0, NP - N), (0, 0))).reshape(NP // 2, 2 * F)
    t2 = _tc_mm(X2, _pair_blockdiag(W0))
    g2, dv2 = _tc_scale(t2, degp2)
    for b, Wn in ((b0, W1), (b1, W2), (b2, W3)):
        A = edge_k(g2.reshape(NP, H), src3, dst3)
        g2 = _tc_layer(A.reshape(NC, NP // 2, 2 * H), g2, dv2,
                       jnp.concatenate([b, b]).reshape(1, 2 * H),
                       _pair_blockdiag(Wn))
    A = edge_k(g2.reshape(NP, H), src3, dst3)
    h2 = _tc_last(A.reshape(NC, NP // 2, 2 * H), g2, dv2,
                  jnp.concatenate([b3, b3]).reshape(1, 2 * H))

    PW = NP // NW
    bidsw = jnp.concatenate(
        [batch_index, jnp.full((NP - N,), G, jnp.int32)]).reshape(NW, PW)
    pool_k = _build_pool_kernel(NP, H, G, PW)
    msp, cntp = pool_k(h2.reshape(NP, H), bidsw)

    return _tc_readout(msp, cntp, Wout, bout)


# speedup vs baseline: 1.0170x; 1.0170x over previous
import functools

import jax
import jax.numpy as jnp
from jax import lax
from jax.experimental import pallas as pl
from jax.experimental.pallas import tpu as pltpu
from jax.experimental.pallas import tpu_sc as plsc

NC = 2
NS = 16
NW = NC * NS
LANES = 16


def _mesh():
    return plsc.VectorSubcoreMesh(core_axis_name="c", subcore_axis_name="s",
                                  num_cores=NC, num_subcores=NS)


@functools.cache
def _build_deg_kernel(NP, C, K, NB=5):
    rps = NP // NS
    ZR = rps // 5
    assert C % NB == 0

    @functools.partial(
        pl.kernel,
        mesh=_mesh(),
        compiler_params=pltpu.CompilerParams(use_tc_tiling_on_sc=False),
        out_type=jax.ShapeDtypeStruct((NC, NP, LANES), jnp.float32),
        scratch_types=[
            pltpu.VMEM((C, K), jnp.int32),
            pltpu.VMEM((K, LANES), jnp.float32),
            pltpu.VMEM((ZR, LANES), jnp.float32),
            pltpu.VMEM_SHARED((NP, LANES), jnp.float32),
        ] + [pltpu.SemaphoreType.DMA] * NB,
    )
    def deg_kernel(dst_hbm, out_hbm, dst_v, ones_v, z_v, acc_sh, *sems):
        cid = lax.axis_index("c")
        sid = lax.axis_index("s")
        wid = cid * NS + sid

        def _fill(i, carry):
            ones_v[i] = jnp.ones((LANES,), jnp.float32)
            return carry
        lax.fori_loop(0, K, _fill, 0)

        def _zrow(i, carry):
            z_v[i] = jnp.zeros((LANES,), jnp.float32)
            return carry
        lax.fori_loop(0, ZR, _zrow, 0)

        def _zcopy(j, carry):
            pltpu.sync_copy(z_v, acc_sh.at[pl.ds(sid * rps + j * ZR, ZR)])
            return carry
        lax.fori_loop(0, rps // ZR, _zcopy, 0)
        plsc.subcore_barrier()

        pltpu.sync_copy(dst_hbm.at[wid], dst_v)

        def _iter(t, carry):
            for b in range(NB):
                j = t * NB + b

                @pl.when(t > 0)
                def _():
                    pltpu.make_async_copy(ones_v, acc_sh.at[dst_v.at[j - NB]],
                                          sems[b]).wait()
                pltpu.async_copy(ones_v, acc_sh.at[dst_v.at[j]], sems[b],
                                 add=True)
            return carry
        lax.fori_loop(0, C // NB, _iter, 0)
        for b in range(NB):
            pltpu.make_async_copy(ones_v, acc_sh.at[dst_v.at[C - NB + b]],
                                  sems[b]).wait()
        plsc.subcore_barrier()

        pltpu.sync_copy(acc_sh.at[pl.ds(sid * rps, rps)],
                        out_hbm.at[cid, pl.ds(sid * rps, rps)])

    return deg_kernel


@functools.cache
def _build_edge_kernel(NP, H, C, K, NB=5):
    rps = NP // NS
    ZR = rps // 5
    assert C % NB == 0

    @functools.partial(
        pl.kernel,
        mesh=_mesh(),
        compiler_params=pltpu.CompilerParams(use_tc_tiling_on_sc=False),
        out_type=jax.ShapeDtypeStruct((NC, NP, H), jnp.float32),
        scratch_types=[
            pltpu.VMEM((C, K), jnp.int32),
            pltpu.VMEM((C, K), jnp.int32),
            pltpu.VMEM((NB, K, H), jnp.float32),
            pltpu.VMEM((ZR, H), jnp.float32),
            pltpu.VMEM_SHARED((NP, H), jnp.float32),
        ] + [pltpu.SemaphoreType.DMA] * NB,
    )
    def edge_kernel(g_hbm, src_hbm, dst_hbm, out_hbm,
                    src_v, dst_v, rows_v, z_v, acc_sh, *gsem):
        cid = lax.axis_index("c")
        sid = lax.axis_index("s")
        wid = cid * NS + sid

        def _zrow(i, carry):
            for j in range(H // LANES):
                z_v[i, pl.ds(j * LANES, LANES)] = jnp.zeros((LANES,), jnp.float32)
            return carry
        lax.fori_loop(0, ZR, _zrow, 0)

        def _zcopy(j, carry):
            pltpu.sync_copy(z_v, acc_sh.at[pl.ds(sid * rps + j * ZR, ZR)])
            return carry
        lax.fori_loop(0, rps // ZR, _zcopy, 0)
        plsc.subcore_barrier()

        pltpu.sync_copy(src_hbm.at[wid], src_v)
        pltpu.sync_copy(dst_hbm.at[wid], dst_v)

        for b in range(NB):
            pltpu.async_copy(g_hbm.at[src_v.at[b]], rows_v.at[b], gsem[b])

        def _iter(t, carry):
            for b in range(NB):
                j = t * NB + b
                pltpu.make_async_copy(g_hbm.at[src_v.at[j]],
                                      rows_v.at[b], gsem[b]).wait()
                pltpu.sync_copy(rows_v.at[b], acc_sh.at[dst_v.at[j]], add=True)
                nxt = j + NB

                @pl.when(nxt < C)
                def _():
                    pltpu.async_copy(g_hbm.at[src_v.at[nxt]],
                                     rows_v.at[b], gsem[b])
            return carry
        lax.fori_loop(0, C // NB, _iter, 0)
        plsc.subcore_barrier()

        pltpu.sync_copy(acc_sh.at[pl.ds(sid * rps, rps)],
                        out_hbm.at[cid, pl.ds(sid * rps, rps)])

    return edge_kernel


@functools.cache
def _build_pool_kernel(N, H, G, PW):
    Gp = G + 1

    @functools.partial(
        pl.kernel,
        mesh=_mesh(),
        compiler_params=pltpu.CompilerParams(use_tc_tiling_on_sc=False),
        out_type=(
            jax.ShapeDtypeStruct((NW, G, 2 * H), jnp.float32),
            jax.ShapeDtypeStruct((NW, G, LANES), jnp.float32),
        ),
        scratch_types=[
            pltpu.VMEM((PW, H), jnp.float32),
            pltpu.VMEM((PW,), jnp.int32),
            pltpu.VMEM((Gp, 2 * H), jnp.float32),
            pltpu.VMEM((Gp, LANES), jnp.float32),
        ],
    )
    def pool_kernel(h_hbm, bids_hbm, oms_hbm, ocnt_hbm,
                    rows_v, bids_v, ms_v, ct_v):
        cid = lax.axis_index("c")
        sid = lax.axis_index("s")
        wid = cid * NS + sid

        def _init(i, carry):
            for j in range(H // LANES):
                ms_v[i, pl.ds(j * LANES, LANES)] = jnp.full((LANES,), -jnp.inf,
                                                            jnp.float32)
                ms_v[i, pl.ds(H + j * LANES, LANES)] = jnp.zeros((LANES,),
                                                                 jnp.float32)
            ct_v[i] = jnp.zeros((LANES,), jnp.float32)
            return carry
        lax.fori_loop(0, Gp, _init, 0)

        pltpu.sync_copy(h_hbm.at[pl.ds(wid * PW, PW)], rows_v)
        pltpu.sync_copy(bids_hbm.at[wid], bids_v)

        def _grp(t, carry):
            bid16 = bids_v[pl.ds(t * LANES, LANES)]
            for j in range(LANES):
                bid = bid16[j]
                k = t * LANES + j
                for q in range(H // LANES):
                    r = rows_v[k, pl.ds(q * LANES, LANES)]
                    m = ms_v[bid, pl.ds(q * LANES, LANES)]
                    ms_v[bid, pl.ds(q * LANES, LANES)] = jnp.maximum(m, r)
                    s = ms_v[bid, pl.ds(H + q * LANES, LANES)]
                    ms_v[bid, pl.ds(H + q * LANES, LANES)] = s + r
                c = ct_v[bid]
                ct_v[bid] = c + 1.0
            return carry
        lax.fori_loop(0, PW // LANES, _grp, 0)

        pltpu.sync_copy(ms_v.at[pl.ds(0, G)], oms_hbm.at[wid])
        pltpu.sync_copy(ct_v.at[pl.ds(0, G)], ocnt_hbm.at[wid])

    return pool_kernel


def _tc_first(X2, W02, degp2, BN2=640):
    NP2, F2 = X2.shape
    H2 = W02.shape[1]

    def body(x_ref, w_ref, dp_ref, g_ref, dv_ref):
        de = lax.rsqrt(jnp.maximum(
            1.0 + dp_ref[0, :, 0:1] + dp_ref[1, :, 0:1], 1.0))
        do = lax.rsqrt(jnp.maximum(
            1.0 + dp_ref[0, :, 16:17] + dp_ref[1, :, 16:17], 1.0))
        dv = jnp.concatenate(
            [jnp.broadcast_to(de, (BN2, H2 // 2)),
             jnp.broadcast_to(do, (BN2, H2 // 2))], axis=1)
        dv_ref[...] = dv
        g_ref[...] = jnp.dot(x_ref[...], w_ref[...],
                             preferred_element_type=jnp.float32) * dv

    return pl.pallas_call(
        body,
        grid=(NP2 // BN2,),
        in_specs=[
            pl.BlockSpec((BN2, F2), lambda i: (i, 0)),
            pl.BlockSpec((F2, H2), lambda i: (0, 0)),
            pl.BlockSpec((NC, BN2, 2 * LANES), lambda i: (0, i, 0)),
        ],
        out_specs=[
            pl.BlockSpec((BN2, H2), lambda i: (i, 0)),
            pl.BlockSpec((BN2, H2), lambda i: (i, 0)),
        ],
        out_shape=[
            jax.ShapeDtypeStruct((NP2, H2), jnp.float32),
            jax.ShapeDtypeStruct((NP2, H2), jnp.float32),
        ],
    )(X2, W02, degp2)


def _tc_layer(A2, g2, dv2, b2, W2n, BN2=640):
    NP2, H2 = g2.shape

    def body(a_ref, g_ref, dv_ref, b_ref, w_ref, o_ref):
        dv = dv_ref[...]
        h = jnp.tanh(dv * (a_ref[0] + a_ref[1] + g_ref[...]) + b_ref[...])
        o_ref[...] = jnp.dot(h, w_ref[...],
                             preferred_element_type=jnp.float32) * dv

    return pl.pallas_call(
        body,
        grid=(NP2 // BN2,),
        in_specs=[
            pl.BlockSpec((NC, BN2, H2), lambda i: (0, i, 0)),
            pl.BlockSpec((BN2, H2), lambda i: (i, 0)),
            pl.BlockSpec((BN2, H2), lambda i: (i, 0)),
            pl.BlockSpec((1, H2), lambda i: (0, 0)),
            pl.BlockSpec((H2, H2), lambda i: (0, 0)),
        ],
        out_specs=pl.BlockSpec((BN2, H2), lambda i: (i, 0)),
        out_shape=jax.ShapeDtypeStruct((NP2, H2), jnp.float32),
    )(A2, g2, dv2, b2, W2n)


def _tc_last(A2, g2, dv2, b2, BN2=640):
    NP2, H2 = g2.shape

    def body(a_ref, g_ref, dv_ref, b_ref, o_ref):
        o_ref[...] = jnp.tanh(
            dv_ref[...] * (a_ref[0] + a_ref[1] + g_ref[...]) + b_ref[...])

    return pl.pallas_call(
        body,
        grid=(NP2 // BN2,),
        in_specs=[
            pl.BlockSpec((NC, BN2, H2), lambda i: (0, i, 0)),
            pl.BlockSpec((BN2, H2), lambda i: (i, 0)),
            pl.BlockSpec((BN2, H2), lambda i: (i, 0)),
            pl.BlockSpec((1, H2), lambda i: (0, 0)),
        ],
        out_specs=pl.BlockSpec((BN2, H2), lambda i: (i, 0)),
        out_shape=jax.ShapeDtypeStruct((NP2, H2), jnp.float32),
    )(A2, g2, dv2, b2)


def _tc_readout(msp, cntp, Wout, bout):
    G = msp.shape[1]
    H = msp.shape[2] // 2

    def body(ms_ref, ct_ref, w_ref, b_ref, out_ref, hid_ref):
        mx = jnp.max(ms_ref[:, :, :H], axis=0)
        mx = jnp.where(mx == -jnp.inf, 0.0, mx)
        sm = jnp.sum(ms_ref[:, :, H:], axis=0)
        ct = jnp.sum(ct_ref[:, :, 0], axis=0)
        mean = sm / jnp.maximum(ct, 1.0)[:, None]
        hidden = jnp.concatenate([mx, mean], axis=1)
        hid_ref[...] = hidden
        out_ref[...] = jnp.dot(hidden, w_ref[...],
                               preferred_element_type=jnp.float32) + b_ref[...]

    return pl.pallas_call(
        body,
        out_shape=(
            jax.ShapeDtypeStruct((G, 1), jnp.float32),
            jax.ShapeDtypeStruct((G, 2 * H), jnp.float32),
        ),
    )(msp, cntp, Wout, bout.reshape(1, 1))


def _pair_blockdiag(W):
    Fi, Fo = W.shape
    Z = jnp.zeros((Fi, Fo), W.dtype)
    return jnp.concatenate(
        [jnp.concatenate([W, Z], axis=1), jnp.concatenate([Z, W], axis=1)],
        axis=0)


def kernel(x, edge_index, batch_index, W0, b0, W1, b1, W2, b2, W3, b3,
           Wout, bout):
    N, F = x.shape
    H = W0.shape[1]
    E = edge_index.shape[1]
    G = 256
    K = 125
    EW = E // NW
    C = EW // K
    assert EW * NW == E and C * K == EW

    src3 = edge_index[0].reshape(NW, C, K)
    dst3 = edge_index[1].reshape(NW, C, K)

    NP = ((N + 1279) // 1280) * 1280
    deg_k = _build_deg_kernel(NP, C, K)
    edge_k = _build_edge_kernel(NP, H, C, K)

    degp = deg_k(dst3)
    degp2 = degp.reshape(NC, NP // 2, 2 * LANES)
    X2 = jnp.pad(x, ((0, NP - N), (0, 0))).reshape(NP // 2, 2 * F)
    g2, dv2 = _tc_first(X2, _pair_blockdiag(W0), degp2)
    for b, Wn in ((b0, W1), (b1, W2), (b2, W3)):
        A = edge_k(g2.reshape(NP, H), src3, dst3)
        g2 = _tc_layer(A.reshape(NC, NP // 2, 2 * H), g2, dv2,
                       jnp.concatenate([b, b]).reshape(1, 2 * H),
                       _pair_blockdiag(Wn))
    A = edge_k(g2.reshape(NP, H), src3, dst3)
    h2 = _tc_last(A.reshape(NC, NP // 2, 2 * H), g2, dv2,
                  jnp.concatenate([b3, b3]).reshape(1, 2 * H))

    PW = NP // NW
    bidsw = jnp.concatenate(
        [batch_index, jnp.full((NP - N,), G, jnp.int32)]).reshape(NW, PW)
    pool_k = _build_pool_kernel(NP, H, G, PW)
    msp, cntp = pool_k(h2.reshape(NP, H), bidsw)

    return _tc_readout(msp, cntp, Wout, bout)
